# baseline (speedup 1.0000x reference)
import functools

import jax
import jax.numpy as jnp
from jax import lax
from jax.experimental import pallas as pl
from jax.experimental.pallas import tpu as pltpu
from jax.experimental.pallas import tpu_sc as plsc

N = 10000
D = 256
E = 160000
BM = 512
NP = 10240
NCORE = 2
NTILE = 16
RPT = NP // NTILE
CHUNK = 128
CHUNK_P = 64
E_PAD = 163840
NCH_DEG = E_PAD // (NCORE * NTILE * CHUNK)
NCH_PROP = E_PAD // (NTILE * CHUNK_P)
HALF = D // 2


@functools.cache
def _sc_mesh():
    return plsc.VectorSubcoreMesh(
        core_axis_name="c", subcore_axis_name="s",
        num_cores=NCORE, num_subcores=NTILE)


def _deg_body(src_hbm, dst_hbm, z1_hbm, cnt_hbm, loop_hbm,
              srcv, dstv, onesv, lbuf, cnt_sh, loop_sh):
    c = lax.axis_index("c")
    s = lax.axis_index("s")
    t = c * NTILE + s
    pltpu.sync_copy(src_hbm.at[t], srcv)
    pltpu.sync_copy(dst_hbm.at[t], dstv)
    r0 = s * RPT
    pltpu.sync_copy(z1_hbm.at[pl.ds(r0, RPT)], cnt_sh.at[pl.ds(r0, RPT)])
    pltpu.sync_copy(z1_hbm.at[pl.ds(r0, RPT)], loop_sh.at[pl.ds(r0, RPT)])
    for k in range(CHUNK // 16):
        onesv[pl.ds(k * 16, 16)] = jnp.full((16,), 1.0, jnp.float32)
    plsc.subcore_barrier()

    @pl.loop(0, NCH_DEG)
    def _chunk(j):
        for k in range(CHUNK // 16):
            sv = srcv[j, pl.ds(k * 16, 16)]
            dv = dstv[j, pl.ds(k * 16, 16)]
            lbuf[pl.ds(k * 16, 16)] = jnp.where(sv == dv, 1.0, 0.0)
        pltpu.sync_copy(onesv, cnt_sh.at[dstv.at[j]], add=True)
        pltpu.sync_copy(lbuf, loop_sh.at[dstv.at[j]], add=True)

    plsc.subcore_barrier()
    pltpu.sync_copy(cnt_sh.at[pl.ds(r0, RPT)], cnt_hbm.at[c, pl.ds(r0, RPT)])
    pltpu.sync_copy(loop_sh.at[pl.ds(r0, RPT)], loop_hbm.at[c, pl.ds(r0, RPT)])


@functools.cache
def _deg_call():
    return pl.kernel(
        _deg_body,
        out_type=[jax.ShapeDtypeStruct((NCORE, NP), jnp.float32),
                  jax.ShapeDtypeStruct((NCORE, NP), jnp.float32)],
        mesh=_sc_mesh(),
        scratch_types=[
            pltpu.VMEM((NCH_DEG, CHUNK), jnp.int32),
            pltpu.VMEM((NCH_DEG, CHUNK), jnp.int32),
            pltpu.VMEM((CHUNK,), jnp.float32),
            pltpu.VMEM((CHUNK,), jnp.float32),
            pltpu.VMEM_SHARED((NP,), jnp.float32),
            pltpu.VMEM_SHARED((NP,), jnp.float32),
        ],
    )


def _prop_body(y_hbm, src_hbm, dst_hbm, z2_hbm, out_hbm,
               idxv, dstv, buf0, buf1, sem0, sem1, sems0, sems1, acc_sh):
    c = lax.axis_index("c")
    s = lax.axis_index("s")
    pltpu.sync_copy(src_hbm.at[s], idxv)
    pltpu.sync_copy(dst_hbm.at[s], dstv)
    r0 = s * RPT
    pltpu.sync_copy(z2_hbm.at[pl.ds(r0, RPT)], acc_sh.at[pl.ds(r0, RPT)])
    plsc.subcore_barrier()

    coff = c * HALF

    def _start(row, half, buf, sem):
        pltpu.async_copy(
            y_hbm.at[idxv.at[row, pl.ds(half * CHUNK_P, CHUNK_P)],
                     pl.ds(coff, HALF)],
            buf, sem)

    def _wait(row, half, buf, sem):
        pltpu.make_async_copy(
            y_hbm.at[idxv.at[row, pl.ds(half * CHUNK_P, CHUNK_P)],
                     pl.ds(coff, HALF)],
            buf, sem).wait()

    def _sstart(j, buf, sem):
        pltpu.async_copy(buf, acc_sh.at[dstv.at[j]], sem, add=True)

    def _swait(j, buf, sem):
        pltpu.make_async_copy(buf, acc_sh.at[dstv.at[j]], sem).wait()

    _start(0, 0, buf0, sem0)

    @pl.loop(0, NCH_PROP // 2)
    def _pair(g):
        j0 = g * 2
        _wait(g, 0, buf0, sem0)
        _sstart(j0, buf0, sems0)

        @pl.when(g > 0)
        def _():
            _swait(j0 - 1, buf1, sems1)

        _start(g, 1, buf1, sem1)
        _wait(g, 1, buf1, sem1)
        _sstart(j0 + 1, buf1, sems1)
        _swait(j0, buf0, sems0)

        @pl.when(g < NCH_PROP // 2 - 1)
        def _():
            _start(g + 1, 0, buf0, sem0)

    _swait(NCH_PROP - 1, buf1, sems1)

    plsc.subcore_barrier()
    pltpu.sync_copy(acc_sh.at[pl.ds(r0, RPT)], out_hbm.at[c, pl.ds(r0, RPT)])


@functools.cache
def _prop_call():
    return pl.kernel(
        _prop_body,
        out_type=jax.ShapeDtypeStruct((NCORE, NP, HALF), jnp.float32),
        mesh=_sc_mesh(),
        scratch_types=[
            pltpu.VMEM((NCH_PROP // 2, 2 * CHUNK_P), jnp.int32),
            pltpu.VMEM((NCH_PROP, CHUNK_P), jnp.int32),
            pltpu.VMEM((CHUNK_P, HALF), jnp.float32),
            pltpu.VMEM((CHUNK_P, HALF), jnp.float32),
            pltpu.SemaphoreType.DMA,
            pltpu.SemaphoreType.DMA,
            pltpu.SemaphoreType.DMA,
            pltpu.SemaphoreType.DMA,
            pltpu.VMEM_SHARED((NP, HALF), jnp.float32),
        ],
    )


def _norm(cnt_ref, loop_ref):
    cnt = cnt_ref[0] + cnt_ref[1]
    lc = loop_ref[0] + loop_ref[1]
    wl = jnp.where(lc == 0.0, 1.0, 0.0)
    deg = cnt + wl
    d = lax.rsqrt(deg)
    return d, d * d * wl, jnp.sqrt(deg)


def _scale1_body(x_ref, w_ref, cnt_ref, loop_ref, y_ref):
    xw = jnp.dot(x_ref[...], w_ref[...], preferred_element_type=jnp.float32)
    d, _, _ = _norm(cnt_ref, loop_ref)
    y_ref[...] = xw * d


def _mid_body(acc_ref, y1_ref, cnt_ref, loop_ref, b_ref, w2_ref, y_ref):
    d, dw, dinv = _norm(cnt_ref, loop_ref)
    agg = jnp.concatenate([acc_ref[0], acc_ref[1]], axis=1)
    xw = y1_ref[...] * dinv
    h = agg * d + xw * dw + b_ref[...]
    h = jnp.maximum(h, 0.0)
    hw = jnp.dot(h, w2_ref[...], preferred_element_type=jnp.float32)
    y_ref[...] = hw * d


def _final_body(acc_ref, y2_ref, cnt_ref, loop_ref, b_ref, out_ref):
    d, dw, dinv = _norm(cnt_ref, loop_ref)
    agg = jnp.concatenate([acc_ref[0], acc_ref[1]], axis=1)
    o = agg * d + (y2_ref[...] * dinv) * dw + b_ref[...]
    m = jnp.max(o, axis=1, keepdims=True)
    e = jnp.exp(o - m)
    out_ref[...] = e / jnp.sum(e, axis=1, keepdims=True)


_spec_rows = pl.BlockSpec((BM, D), lambda i: (i, 0))
_spec_w = pl.BlockSpec((D, D), lambda i: (0, 0))
_spec_nrm = pl.BlockSpec((2, BM, 1), lambda i: (0, i, 0))
_spec_cat = pl.BlockSpec((2, BM, HALF), lambda i: (0, i, 0))
_spec_b = pl.BlockSpec((1, D), lambda i: (0, 0))

_scale1 = pl.pallas_call(
    _scale1_body,
    grid=(NP // BM,),
    in_specs=[_spec_rows, _spec_w, _spec_nrm, _spec_nrm],
    out_specs=_spec_rows,
    out_shape=jax.ShapeDtypeStruct((NP, D), jnp.float32),
)

_mid = pl.pallas_call(
    _mid_body,
    grid=(NP // BM,),
    in_specs=[_spec_cat, _spec_rows, _spec_nrm, _spec_nrm, _spec_b, _spec_w],
    out_specs=_spec_rows,
    out_shape=jax.ShapeDtypeStruct((NP, D), jnp.float32),
)

_final = pl.pallas_call(
    _final_body,
    grid=(NP // BM,),
    in_specs=[_spec_cat, _spec_rows, _spec_nrm, _spec_nrm, _spec_b],
    out_specs=_spec_rows,
    out_shape=jax.ShapeDtypeStruct((N, D), jnp.float32),
)


@jax.jit
def kernel(x, edge_index, W1, b1, W2, b2):
    src = edge_index[0]
    dst = edge_index[1]
    npad = E_PAD - E
    padr = (jnp.arange(npad, dtype=jnp.int32) % (NP - N)) + N
    srcp = jnp.concatenate([src, padr])
    dstp = jnp.concatenate([dst, padr])
    src_deg = srcp.reshape(NCORE * NTILE, NCH_DEG, CHUNK)
    dst_deg = dstp.reshape(NCORE * NTILE, NCH_DEG, CHUNK)
    src_prop = srcp.reshape(NTILE, NCH_PROP // 2, 2 * CHUNK_P)
    dst_prop = dstp.reshape(NTILE, NCH_PROP, CHUNK_P)
    z1 = jnp.zeros((NP,), jnp.float32)
    z2 = jnp.zeros((NP, HALF), jnp.float32)

    cntp, loopp = _deg_call()(src_deg, dst_deg, z1)
    cnt3 = cntp.reshape(2, NP, 1)
    loop3 = loopp.reshape(2, NP, 1)

    y1 = _scale1(x, W1, cnt3, loop3)
    acc1 = _prop_call()(y1, src_prop, dst_prop, z2)
    y2 = _mid(acc1, y1, cnt3, loop3, b1.reshape(1, D), W2)
    acc2 = _prop_call()(y2, src_prop, dst_prop, z2)
    return _final(acc2, y2, cnt3, loop3, b2.reshape(1, D))


# speedup vs baseline: 1.4041x; 1.4041x over previous
import functools

import jax
import jax.numpy as jnp
from jax import lax
from jax.experimental import pallas as pl
from jax.experimental.pallas import tpu as pltpu
from jax.experimental.pallas import tpu_sc as plsc

N = 10000
D = 256
E = 160000
BM = 512
NP = 10240
NCORE = 2
NTILE = 16
RPT = NP // NTILE
CHUNK = 128
CHUNK_P = 64
E_PAD = 163840
NCH_DEG = E_PAD // (NCORE * NTILE * CHUNK)
NCH_PROP = E_PAD // (NTILE * CHUNK_P)
NCH_S = NCH_PROP // 2
HALF = D // 2


@functools.cache
def _sc_mesh():
    return plsc.VectorSubcoreMesh(
        core_axis_name="c", subcore_axis_name="s",
        num_cores=NCORE, num_subcores=NTILE)


def _deg_body(src_hbm, dst_hbm, z1_hbm, cnt_hbm, loop_hbm,
              srcv, dstv, onesv, lbuf, cnt_sh, loop_sh):
    c = lax.axis_index("c")
    s = lax.axis_index("s")
    t = c * NTILE + s
    pltpu.sync_copy(src_hbm.at[t], srcv)
    pltpu.sync_copy(dst_hbm.at[t], dstv)
    r0 = s * RPT
    pltpu.sync_copy(z1_hbm.at[pl.ds(r0, RPT)], cnt_sh.at[pl.ds(r0, RPT)])
    pltpu.sync_copy(z1_hbm.at[pl.ds(r0, RPT)], loop_sh.at[pl.ds(r0, RPT)])
    for k in range(CHUNK // 16):
        onesv[pl.ds(k * 16, 16)] = jnp.full((16,), 1.0, jnp.float32)
    plsc.subcore_barrier()

    @pl.loop(0, NCH_DEG)
    def _chunk(j):
        for k in range(CHUNK // 16):
            sv = srcv[j, pl.ds(k * 16, 16)]
            dv = dstv[j, pl.ds(k * 16, 16)]
            lbuf[pl.ds(k * 16, 16)] = jnp.where(sv == dv, 1.0, 0.0)
        pltpu.sync_copy(onesv, cnt_sh.at[dstv.at[j]], add=True)
        pltpu.sync_copy(lbuf, loop_sh.at[dstv.at[j]], add=True)

    plsc.subcore_barrier()
    pltpu.sync_copy(cnt_sh.at[pl.ds(r0, RPT)], cnt_hbm.at[c, pl.ds(r0, RPT)])
    pltpu.sync_copy(loop_sh.at[pl.ds(r0, RPT)], loop_hbm.at[c, pl.ds(r0, RPT)])


@functools.cache
def _deg_call():
    return pl.kernel(
        _deg_body,
        out_type=[jax.ShapeDtypeStruct((NCORE, NP), jnp.float32),
                  jax.ShapeDtypeStruct((NCORE, NP), jnp.float32)],
        mesh=_sc_mesh(),
        scratch_types=[
            pltpu.VMEM((NCH_DEG, CHUNK), jnp.int32),
            pltpu.VMEM((NCH_DEG, CHUNK), jnp.int32),
            pltpu.VMEM((CHUNK,), jnp.float32),
            pltpu.VMEM((CHUNK,), jnp.float32),
            pltpu.VMEM_SHARED((NP,), jnp.float32),
            pltpu.VMEM_SHARED((NP,), jnp.float32),
        ],
    )


def _prop_body(y_hbm, src_hbm, dst_hbm, z2_hbm, out_hbm,
               idxv, dstv, bufa, bufb, sema, semb, semi0, semi1, acc_sh):
    c = lax.axis_index("c")
    s = lax.axis_index("s")
    pltpu.sync_copy(src_hbm.at[s], idxv)
    r0 = s * RPT
    pltpu.sync_copy(z2_hbm.at[pl.ds(r0, RPT)], acc_sh.at[pl.ds(r0, RPT)])
    plsc.subcore_barrier()

    coff = c * HALF

    def _startg(g, buf, sem):
        for h in (0, 1):
            pltpu.async_copy(
                y_hbm.at[idxv.at[g, pl.ds(h * CHUNK_P, CHUNK_P)],
                         pl.ds(coff, HALF)],
                buf.at[pl.ds(h * CHUNK_P, CHUNK_P)], sem)

    def _waitg(g, buf, sem):
        for h in (0, 1):
            pltpu.make_async_copy(
                y_hbm.at[idxv.at[g, pl.ds(h * CHUNK_P, CHUNK_P)],
                         pl.ds(coff, HALF)],
                buf.at[pl.ds(h * CHUNK_P, CHUNK_P)], sem).wait()

    def _starti(g, p, sem):
        pltpu.async_copy(dst_hbm.at[s, g], dstv.at[p], sem)

    def _waiti(g, p, sem):
        pltpu.make_async_copy(dst_hbm.at[s, g], dstv.at[p], sem).wait()

    _starti(0, 0, semi0)
    _startg(0, bufa, sema)

    @pl.loop(0, NCH_S // 2)
    def _pair(gg):
        g0 = 2 * gg
        _starti(g0 + 1, 1, semi1)
        _startg(g0 + 1, bufb, semb)
        _waitg(g0, bufa, sema)
        _waiti(g0, 0, semi0)
        pltpu.sync_copy(bufa, acc_sh.at[dstv.at[0]], add=True)

        @pl.when(gg < NCH_S // 2 - 1)
        def _():
            _starti(g0 + 2, 0, semi0)
            _startg(g0 + 2, bufa, sema)

        _waitg(g0 + 1, bufb, semb)
        _waiti(g0 + 1, 1, semi1)
        pltpu.sync_copy(bufb, acc_sh.at[dstv.at[1]], add=True)

    plsc.subcore_barrier()
    pltpu.sync_copy(acc_sh.at[pl.ds(r0, RPT)], out_hbm.at[c, pl.ds(r0, RPT)])


@functools.cache
def _prop_call():
    return pl.kernel(
        _prop_body,
        out_type=jax.ShapeDtypeStruct((NCORE, NP, HALF), jnp.float32),
        mesh=_sc_mesh(),
        scratch_types=[
            pltpu.VMEM((NCH_S, 2 * CHUNK_P), jnp.int32),
            pltpu.VMEM((2, 2 * CHUNK_P), jnp.int32),
            pltpu.VMEM((2 * CHUNK_P, HALF), jnp.float32),
            pltpu.VMEM((2 * CHUNK_P, HALF), jnp.float32),
            pltpu.SemaphoreType.DMA,
            pltpu.SemaphoreType.DMA,
            pltpu.SemaphoreType.DMA,
            pltpu.SemaphoreType.DMA,
            pltpu.VMEM_SHARED((NP, HALF), jnp.float32),
        ],
    )


def _norm(cnt_ref, loop_ref):
    cnt = cnt_ref[0] + cnt_ref[1]
    lc = loop_ref[0] + loop_ref[1]
    wl = jnp.where(lc == 0.0, 1.0, 0.0)
    deg = cnt + wl
    d = lax.rsqrt(deg)
    return d, d * d * wl, jnp.sqrt(deg)


def _scale1_body(x_ref, w_ref, cnt_ref, loop_ref, y_ref):
    xw = jnp.dot(x_ref[...], w_ref[...], preferred_element_type=jnp.float32)
    d, _, _ = _norm(cnt_ref, loop_ref)
    y_ref[...] = xw * d


def _mid_body(acc_ref, y1_ref, cnt_ref, loop_ref, b_ref, w2_ref, y_ref):
    d, dw, dinv = _norm(cnt_ref, loop_ref)
    agg = jnp.concatenate([acc_ref[0], acc_ref[1]], axis=1)
    xw = y1_ref[...] * dinv
    h = agg * d + xw * dw + b_ref[...]
    h = jnp.maximum(h, 0.0)
    hw = jnp.dot(h, w2_ref[...], preferred_element_type=jnp.float32)
    y_ref[...] = hw * d


def _final_body(acc_ref, y2_ref, cnt_ref, loop_ref, b_ref, out_ref):
    d, dw, dinv = _norm(cnt_ref, loop_ref)
    agg = jnp.concatenate([acc_ref[0], acc_ref[1]], axis=1)
    o = agg * d + (y2_ref[...] * dinv) * dw + b_ref[...]
    m = jnp.max(o, axis=1, keepdims=True)
    e = jnp.exp(o - m)
    out_ref[...] = e / jnp.sum(e, axis=1, keepdims=True)


_spec_rows = pl.BlockSpec((BM, D), lambda i: (i, 0))
_spec_w = pl.BlockSpec((D, D), lambda i: (0, 0))
_spec_nrm = pl.BlockSpec((2, BM, 1), lambda i: (0, i, 0))
_spec_cat = pl.BlockSpec((2, BM, HALF), lambda i: (0, i, 0))
_spec_b = pl.BlockSpec((1, D), lambda i: (0, 0))

_scale1 = pl.pallas_call(
    _scale1_body,
    grid=(NP // BM,),
    in_specs=[_spec_rows, _spec_w, _spec_nrm, _spec_nrm],
    out_specs=_spec_rows,
    out_shape=jax.ShapeDtypeStruct((NP, D), jnp.float32),
)

_mid = pl.pallas_call(
    _mid_body,
    grid=(NP // BM,),
    in_specs=[_spec_cat, _spec_rows, _spec_nrm, _spec_nrm, _spec_b, _spec_w],
    out_specs=_spec_rows,
    out_shape=jax.ShapeDtypeStruct((NP, D), jnp.float32),
)

_final = pl.pallas_call(
    _final_body,
    grid=(NP // BM,),
    in_specs=[_spec_cat, _spec_rows, _spec_nrm, _spec_nrm, _spec_b],
    out_specs=_spec_rows,
    out_shape=jax.ShapeDtypeStruct((N, D), jnp.float32),
)


@jax.jit
def kernel(x, edge_index, W1, b1, W2, b2):
    src = edge_index[0]
    dst = edge_index[1]
    npad = E_PAD - E
    padr = (jnp.arange(npad, dtype=jnp.int32) % (NP - N)) + N
    srcp = jnp.concatenate([src, padr])
    dstp = jnp.concatenate([dst, padr])
    src_deg = srcp.reshape(NCORE * NTILE, NCH_DEG, CHUNK)
    dst_deg = dstp.reshape(NCORE * NTILE, NCH_DEG, CHUNK)
    src_prop = srcp.reshape(NTILE, NCH_S, 2 * CHUNK_P)
    dst_prop = dstp.reshape(NTILE, NCH_S, 2 * CHUNK_P)
    z1 = jnp.zeros((NP,), jnp.float32)
    z2 = jnp.zeros((NP, HALF), jnp.float32)

    cntp, loopp = _deg_call()(src_deg, dst_deg, z1)
    cnt3 = cntp.reshape(2, NP, 1)
    loop3 = loopp.reshape(2, NP, 1)

    y1 = _scale1(x, W1, cnt3, loop3)
    acc1 = _prop_call()(y1, src_prop, dst_prop, z2)
    y2 = _mid(acc1, y1, cnt3, loop3, b1.reshape(1, D), W2)
    acc2 = _prop_call()(y2, src_prop, dst_prop, z2)
    return _final(acc2, y2, cnt3, loop3, b2.reshape(1, D))


# speedup vs baseline: 1.4434x; 1.0280x over previous
import functools

import jax
import jax.numpy as jnp
from jax import lax
from jax.experimental import pallas as pl
from jax.experimental.pallas import tpu as pltpu
from jax.experimental.pallas import tpu_sc as plsc

N = 10000
D = 256
E = 160000
BM = 512
NP = 10240
NCORE = 2
NTILE = 16
RPT = NP // NTILE
CHUNK = 128
CHUNK_P = 64
E_PAD = 163840
NCH_DEG = E_PAD // (NCORE * NTILE * CHUNK)
NCH_PROP = E_PAD // (NTILE * CHUNK_P)
NCH_S = NCH_PROP // 2
HALF = D // 2


@functools.cache
def _sc_mesh():
    return plsc.VectorSubcoreMesh(
        core_axis_name="c", subcore_axis_name="s",
        num_cores=NCORE, num_subcores=NTILE)


def _deg_body(src_hbm, dst_hbm, z1_hbm, cnt_hbm, loop_hbm,
              srcv, dstv, onesv, lbuf, cnt_sh, loop_sh):
    c = lax.axis_index("c")
    s = lax.axis_index("s")
    t = c * NTILE + s
    pltpu.sync_copy(src_hbm.at[t], srcv)
    pltpu.sync_copy(dst_hbm.at[t], dstv)
    r0 = s * RPT
    pltpu.sync_copy(z1_hbm.at[pl.ds(r0, RPT)], cnt_sh.at[pl.ds(r0, RPT)])
    pltpu.sync_copy(z1_hbm.at[pl.ds(r0, RPT)], loop_sh.at[pl.ds(r0, RPT)])
    for k in range(CHUNK // 16):
        onesv[pl.ds(k * 16, 16)] = jnp.full((16,), 1.0, jnp.float32)
    plsc.subcore_barrier()

    @pl.loop(0, NCH_DEG)
    def _chunk(j):
        for k in range(CHUNK // 16):
            sv = srcv[j, pl.ds(k * 16, 16)]
            dv = dstv[j, pl.ds(k * 16, 16)]
            lbuf[pl.ds(k * 16, 16)] = jnp.where(sv == dv, 1.0, 0.0)
        pltpu.sync_copy(onesv, cnt_sh.at[dstv.at[j]], add=True)
        pltpu.sync_copy(lbuf, loop_sh.at[dstv.at[j]], add=True)

    plsc.subcore_barrier()
    pltpu.sync_copy(cnt_sh.at[pl.ds(r0, RPT)], cnt_hbm.at[c, pl.ds(r0, RPT)])
    pltpu.sync_copy(loop_sh.at[pl.ds(r0, RPT)], loop_hbm.at[c, pl.ds(r0, RPT)])


@functools.cache
def _deg_call():
    return pl.kernel(
        _deg_body,
        out_type=[jax.ShapeDtypeStruct((NCORE, NP), jnp.float32),
                  jax.ShapeDtypeStruct((NCORE, NP), jnp.float32)],
        mesh=_sc_mesh(),
        scratch_types=[
            pltpu.VMEM((NCH_DEG, CHUNK), jnp.int32),
            pltpu.VMEM((NCH_DEG, CHUNK), jnp.int32),
            pltpu.VMEM((CHUNK,), jnp.float32),
            pltpu.VMEM((CHUNK,), jnp.float32),
            pltpu.VMEM_SHARED((NP,), jnp.float32),
            pltpu.VMEM_SHARED((NP,), jnp.float32),
        ],
    )


def _prop_body(y_hbm, src_hbm, dst_hbm, z2_hbm, out_hbm,
               idxv, dstv, bufa, bufb, sema, semb, semi0, semi1, acc_sh):
    c = lax.axis_index("c")
    s = lax.axis_index("s")
    pltpu.sync_copy(src_hbm.at[s], idxv)
    r0 = s * RPT
    pltpu.sync_copy(z2_hbm.at[pl.ds(r0, RPT)], acc_sh.at[pl.ds(r0, RPT)])
    plsc.subcore_barrier()

    coff = c * HALF

    def _startg(g, buf, sem):
        for h in (0, 1):
            pltpu.async_copy(
                y_hbm.at[idxv.at[g, pl.ds(h * CHUNK_P, CHUNK_P)],
                         pl.ds(coff, HALF)],
                buf.at[pl.ds(h * CHUNK_P, CHUNK_P)], sem)

    def _waitg(g, buf, sem):
        for h in (0, 1):
            pltpu.make_async_copy(
                y_hbm.at[idxv.at[g, pl.ds(h * CHUNK_P, CHUNK_P)],
                         pl.ds(coff, HALF)],
                buf.at[pl.ds(h * CHUNK_P, CHUNK_P)], sem).wait()

    def _starti(g, p, sem):
        pltpu.async_copy(dst_hbm.at[s, g], dstv.at[p], sem)

    def _waiti(g, p, sem):
        pltpu.make_async_copy(dst_hbm.at[s, g], dstv.at[p], sem).wait()

    _starti(0, 0, semi0)
    _startg(0, bufa, sema)

    @pl.loop(0, NCH_S // 2)
    def _pair(gg):
        g0 = 2 * gg
        _starti(g0 + 1, 1, semi1)
        _startg(g0 + 1, bufb, semb)
        _waitg(g0, bufa, sema)
        _waiti(g0, 0, semi0)
        pltpu.sync_copy(bufa, acc_sh.at[dstv.at[0]], add=True)

        @pl.when(gg < NCH_S // 2 - 1)
        def _():
            _starti(g0 + 2, 0, semi0)
            _startg(g0 + 2, bufa, sema)

        _waitg(g0 + 1, bufb, semb)
        _waiti(g0 + 1, 1, semi1)
        pltpu.sync_copy(bufb, acc_sh.at[dstv.at[1]], add=True)

    plsc.subcore_barrier()
    pltpu.sync_copy(acc_sh.at[pl.ds(r0, RPT)], out_hbm.at[c, pl.ds(r0, RPT)])


@functools.cache
def _prop_call():
    return pl.kernel(
        _prop_body,
        out_type=jax.ShapeDtypeStruct((NCORE, NP, HALF), jnp.float32),
        mesh=_sc_mesh(),
        scratch_types=[
            pltpu.VMEM((NCH_S, 2 * CHUNK_P), jnp.int32),
            pltpu.VMEM((2, 2 * CHUNK_P), jnp.int32),
            pltpu.VMEM((2 * CHUNK_P, HALF), jnp.float32),
            pltpu.VMEM((2 * CHUNK_P, HALF), jnp.float32),
            pltpu.SemaphoreType.DMA,
            pltpu.SemaphoreType.DMA,
            pltpu.SemaphoreType.DMA,
            pltpu.SemaphoreType.DMA,
            pltpu.VMEM_SHARED((NP, HALF), jnp.float32),
        ],
    )


def _mm1_body(x_ref, w_ref, xw_ref):
    xw_ref[...] = jnp.dot(x_ref[...], w_ref[...],
                          preferred_element_type=jnp.float32)


def _scale_body(xw_ref, cnt_ref, loop_ref, eye_ref, y_ref, d_ref, dw_ref,
                di_ref):
    eye = eye_ref[...]
    cnt = lax.dot_general(eye, cnt_ref[0:1] + cnt_ref[1:2],
                          (((1,), (1,)), ((), ())),
                          preferred_element_type=jnp.float32)
    lc = lax.dot_general(eye, loop_ref[0:1] + loop_ref[1:2],
                         (((1,), (1,)), ((), ())),
                         preferred_element_type=jnp.float32)
    wl = jnp.where(lc == 0.0, 1.0, 0.0)
    deg = cnt + wl
    d = lax.rsqrt(deg)
    d_ref[...] = d
    dw_ref[...] = d * d * wl
    di_ref[...] = jnp.sqrt(deg)
    y_ref[...] = xw_ref[...] * d


def _mid_body(acc_ref, y1_ref, d_ref, dw_ref, di_ref, b_ref, w2_ref, y_ref):
    d = d_ref[...]
    agg = jnp.concatenate([acc_ref[0], acc_ref[1]], axis=1)
    xw = y1_ref[...] * di_ref[...]
    h = agg * d + xw * dw_ref[...] + b_ref[...]
    h = jnp.maximum(h, 0.0)
    hw = jnp.dot(h, w2_ref[...], preferred_element_type=jnp.float32)
    y_ref[...] = hw * d


def _final_body(acc_ref, y2_ref, d_ref, dw_ref, di_ref, b_ref, out_ref):
    agg = jnp.concatenate([acc_ref[0], acc_ref[1]], axis=1)
    o = (agg * d_ref[...]
         + (y2_ref[...] * di_ref[...]) * dw_ref[...] + b_ref[...])
    m = jnp.max(o, axis=1, keepdims=True)
    e = jnp.exp(o - m)
    out_ref[...] = e / jnp.sum(e, axis=1, keepdims=True)


_spec_rows = pl.BlockSpec((BM, D), lambda i: (i, 0))
_spec_w = pl.BlockSpec((D, D), lambda i: (0, 0))
_spec_hist = pl.BlockSpec((2, BM), lambda i: (0, i))
_spec_col = pl.BlockSpec((BM, 1), lambda i: (i, 0))
_spec_cat = pl.BlockSpec((2, BM, HALF), lambda i: (0, i, 0))
_spec_b = pl.BlockSpec((1, D), lambda i: (0, 0))
_spec_eye = pl.BlockSpec((BM, BM), lambda i: (0, 0))

_col_shape = jax.ShapeDtypeStruct((NP, 1), jnp.float32)

_mm1 = pl.pallas_call(
    _mm1_body,
    grid=(NP // BM,),
    in_specs=[_spec_rows, _spec_w],
    out_specs=_spec_rows,
    out_shape=jax.ShapeDtypeStruct((NP, D), jnp.float32),
)

_scale = pl.pallas_call(
    _scale_body,
    grid=(NP // BM,),
    in_specs=[_spec_rows, _spec_hist, _spec_hist, _spec_eye],
    out_specs=[_spec_rows, _spec_col, _spec_col, _spec_col],
    out_shape=[jax.ShapeDtypeStruct((NP, D), jnp.float32),
               _col_shape, _col_shape, _col_shape],
)

_mid = pl.pallas_call(
    _mid_body,
    grid=(NP // BM,),
    in_specs=[_spec_cat, _spec_rows, _spec_col, _spec_col, _spec_col,
              _spec_b, _spec_w],
    out_specs=_spec_rows,
    out_shape=jax.ShapeDtypeStruct((NP, D), jnp.float32),
)

_final = pl.pallas_call(
    _final_body,
    grid=(NP // BM,),
    in_specs=[_spec_cat, _spec_rows, _spec_col, _spec_col, _spec_col,
              _spec_b],
    out_specs=_spec_rows,
    out_shape=jax.ShapeDtypeStruct((N, D), jnp.float32),
)


@jax.jit
def kernel(x, edge_index, W1, b1, W2, b2):
    src = edge_index[0]
    dst = edge_index[1]
    npad = E_PAD - E
    padr = (jnp.arange(npad, dtype=jnp.int32) % (NP - N)) + N
    srcp = jnp.concatenate([src, padr])
    dstp = jnp.concatenate([dst, padr])
    src_deg = srcp.reshape(NCORE * NTILE, NCH_DEG, CHUNK)
    dst_deg = dstp.reshape(NCORE * NTILE, NCH_DEG, CHUNK)
    src_prop = srcp.reshape(NTILE, NCH_S, 2 * CHUNK_P)
    dst_prop = dstp.reshape(NTILE, NCH_S, 2 * CHUNK_P)
    z1 = jnp.zeros((NP,), jnp.float32)
    z2 = jnp.zeros((NP, HALF), jnp.float32)

    cntp, loopp = _deg_call()(src_deg, dst_deg, z1)

    xw1 = _mm1(x, W1)
    y1, d3, dw3, di3 = _scale(xw1, cntp, loopp, jnp.eye(BM, dtype=jnp.float32))
    acc1 = _prop_call()(y1, src_prop, dst_prop, z2)
    y2 = _mid(acc1, y1, d3, dw3, di3, b1.reshape(1, D), W2)
    acc2 = _prop_call()(y2, src_prop, dst_prop, z2)
    return _final(acc2, y2, d3, dw3, di3, b2.reshape(1, D))


# speedup vs baseline: 1.4475x; 1.0028x over previous
import functools

import jax
import jax.numpy as jnp
from jax import lax
from jax.experimental import pallas as pl
from jax.experimental.pallas import tpu as pltpu
from jax.experimental.pallas import tpu_sc as plsc

N = 10000
D = 256
E = 160000
BM = 512
NP = 10240
NCORE = 2
NTILE = 16
RPT = NP // NTILE
CHUNK = 128
CHUNK_P = 64
E_PAD = 163840
NCH_DEG = E_PAD // (NCORE * NTILE * CHUNK)
NCH_PROP = E_PAD // (NTILE * CHUNK_P)
NCH_S = NCH_PROP // 2
HALF = D // 2


@functools.cache
def _sc_mesh():
    return plsc.VectorSubcoreMesh(
        core_axis_name="c", subcore_axis_name="s",
        num_cores=NCORE, num_subcores=NTILE)


def _deg_body(src_hbm, dst_hbm, z1_hbm, cnt_hbm, loop_hbm,
              srcv, dstv, onesv, lbuf, cnt_sh, loop_sh):
    c = lax.axis_index("c")
    s = lax.axis_index("s")
    t = c * NTILE + s
    pltpu.sync_copy(src_hbm.at[t // 2, pl.ds((t % 2) * NCH_DEG, NCH_DEG)], srcv)
    pltpu.sync_copy(dst_hbm.at[t // 2, pl.ds((t % 2) * NCH_DEG, NCH_DEG)], dstv)
    r0 = s * RPT
    pltpu.sync_copy(z1_hbm.at[pl.ds(r0, RPT)], cnt_sh.at[pl.ds(r0, RPT)])
    pltpu.sync_copy(z1_hbm.at[pl.ds(r0, RPT)], loop_sh.at[pl.ds(r0, RPT)])
    for k in range(CHUNK // 16):
        onesv[pl.ds(k * 16, 16)] = jnp.full((16,), 1.0, jnp.float32)
    plsc.subcore_barrier()

    @pl.loop(0, NCH_DEG)
    def _chunk(j):
        for k in range(CHUNK // 16):
            sv = srcv[j, pl.ds(k * 16, 16)]
            dv = dstv[j, pl.ds(k * 16, 16)]
            lbuf[pl.ds(k * 16, 16)] = jnp.where(sv == dv, 1.0, 0.0)
        pltpu.sync_copy(onesv, cnt_sh.at[dstv.at[j]], add=True)
        pltpu.sync_copy(lbuf, loop_sh.at[dstv.at[j]], add=True)

    plsc.subcore_barrier()
    pltpu.sync_copy(cnt_sh.at[pl.ds(r0, RPT)], cnt_hbm.at[c, pl.ds(r0, RPT)])
    pltpu.sync_copy(loop_sh.at[pl.ds(r0, RPT)], loop_hbm.at[c, pl.ds(r0, RPT)])


@functools.cache
def _deg_call():
    return pl.kernel(
        _deg_body,
        out_type=[jax.ShapeDtypeStruct((NCORE, NP), jnp.float32),
                  jax.ShapeDtypeStruct((NCORE, NP), jnp.float32)],
        mesh=_sc_mesh(),
        scratch_types=[
            pltpu.VMEM((NCH_DEG, CHUNK), jnp.int32),
            pltpu.VMEM((NCH_DEG, CHUNK), jnp.int32),
            pltpu.VMEM((CHUNK,), jnp.float32),
            pltpu.VMEM((CHUNK,), jnp.float32),
            pltpu.VMEM_SHARED((NP,), jnp.float32),
            pltpu.VMEM_SHARED((NP,), jnp.float32),
        ],
    )


def _prop_body(y_hbm, src_hbm, dst_hbm, z2_hbm, out_hbm,
               idxv, dstv, bufa, bufb, sema, semb, semi0, semi1, acc_sh):
    c = lax.axis_index("c")
    s = lax.axis_index("s")
    pltpu.sync_copy(src_hbm.at[s], idxv)
    r0 = s * RPT
    pltpu.sync_copy(z2_hbm.at[pl.ds(r0, RPT)], acc_sh.at[pl.ds(r0, RPT)])
    plsc.subcore_barrier()

    coff = c * HALF

    def _startg(g, buf, sem):
        pltpu.async_copy(
            y_hbm.at[idxv.at[g], pl.ds(coff, HALF)], buf, sem)

    def _waitg(g, buf, sem):
        pltpu.make_async_copy(
            y_hbm.at[idxv.at[g], pl.ds(coff, HALF)], buf, sem).wait()

    def _starti(g, p, sem):
        pltpu.async_copy(dst_hbm.at[s, g], dstv.at[p], sem)

    def _waiti(g, p, sem):
        pltpu.make_async_copy(dst_hbm.at[s, g], dstv.at[p], sem).wait()

    _starti(0, 0, semi0)
    _startg(0, bufa, sema)

    @pl.loop(0, NCH_S // 2)
    def _pair(gg):
        g0 = 2 * gg
        _starti(g0 + 1, 1, semi1)
        _startg(g0 + 1, bufb, semb)
        _waitg(g0, bufa, sema)
        _waiti(g0, 0, semi0)
        pltpu.sync_copy(bufa, acc_sh.at[dstv.at[0]], add=True)

        @pl.when(gg < NCH_S // 2 - 1)
        def _():
            _starti(g0 + 2, 0, semi0)
            _startg(g0 + 2, bufa, sema)

        _waitg(g0 + 1, bufb, semb)
        _waiti(g0 + 1, 1, semi1)
        pltpu.sync_copy(bufb, acc_sh.at[dstv.at[1]], add=True)

    plsc.subcore_barrier()
    pltpu.sync_copy(acc_sh.at[pl.ds(r0, RPT)], out_hbm.at[c, pl.ds(r0, RPT)])


@functools.cache
def _prop_call():
    return pl.kernel(
        _prop_body,
        out_type=jax.ShapeDtypeStruct((NCORE, NP, HALF), jnp.float32),
        mesh=_sc_mesh(),
        scratch_types=[
            pltpu.VMEM((NCH_S, 2 * CHUNK_P), jnp.int32),
            pltpu.VMEM((2, 2 * CHUNK_P), jnp.int32),
            pltpu.VMEM((2 * CHUNK_P, HALF), jnp.float32),
            pltpu.VMEM((2 * CHUNK_P, HALF), jnp.float32),
            pltpu.SemaphoreType.DMA,
            pltpu.SemaphoreType.DMA,
            pltpu.SemaphoreType.DMA,
            pltpu.SemaphoreType.DMA,
            pltpu.VMEM_SHARED((NP, HALF), jnp.float32),
        ],
    )


def _mm1_body(x_ref, w_ref, xw_ref):
    xw_ref[...] = jnp.dot(x_ref[...], w_ref[...],
                          preferred_element_type=jnp.float32)


def _scale_body(xw_ref, cnt_ref, loop_ref, eye_ref, y_ref, d_ref, dw_ref,
                di_ref):
    eye = eye_ref[...]
    cnt = lax.dot_general(eye, cnt_ref[0:1] + cnt_ref[1:2],
                          (((1,), (1,)), ((), ())),
                          preferred_element_type=jnp.float32)
    lc = lax.dot_general(eye, loop_ref[0:1] + loop_ref[1:2],
                         (((1,), (1,)), ((), ())),
                         preferred_element_type=jnp.float32)
    wl = jnp.where(lc == 0.0, 1.0, 0.0)
    deg = cnt + wl
    d = lax.rsqrt(deg)
    d_ref[...] = d
    dw_ref[...] = d * d * wl
    di_ref[...] = jnp.sqrt(deg)
    y_ref[...] = xw_ref[...] * d


def _mid_body(acc_ref, y1_ref, d_ref, dw_ref, di_ref, b_ref, w2_ref, y_ref):
    d = d_ref[...]
    agg = jnp.concatenate([acc_ref[0], acc_ref[1]], axis=1)
    xw = y1_ref[...] * di_ref[...]
    h = agg * d + xw * dw_ref[...] + b_ref[...]
    h = jnp.maximum(h, 0.0)
    hw = jnp.dot(h, w2_ref[...], preferred_element_type=jnp.float32)
    y_ref[...] = hw * d


def _final_body(acc_ref, y2_ref, d_ref, dw_ref, di_ref, b_ref, out_ref):
    agg = jnp.concatenate([acc_ref[0], acc_ref[1]], axis=1)
    o = (agg * d_ref[...]
         + (y2_ref[...] * di_ref[...]) * dw_ref[...] + b_ref[...])
    m = jnp.max(o, axis=1, keepdims=True)
    e = jnp.exp(o - m)
    out_ref[...] = e / jnp.sum(e, axis=1, keepdims=True)


_spec_rows = pl.BlockSpec((BM, D), lambda i: (i, 0))
_spec_w = pl.BlockSpec((D, D), lambda i: (0, 0))
_spec_hist = pl.BlockSpec((2, BM), lambda i: (0, i))
_spec_col = pl.BlockSpec((BM, 1), lambda i: (i, 0))
_spec_cat = pl.BlockSpec((2, BM, HALF), lambda i: (0, i, 0))
_spec_b = pl.BlockSpec((1, D), lambda i: (0, 0))
_spec_eye = pl.BlockSpec((BM, BM), lambda i: (0, 0))

_col_shape = jax.ShapeDtypeStruct((NP, 1), jnp.float32)

_mm1 = pl.pallas_call(
    _mm1_body,
    grid=(NP // BM,),
    in_specs=[_spec_rows, _spec_w],
    out_specs=_spec_rows,
    out_shape=jax.ShapeDtypeStruct((NP, D), jnp.float32),
)

_scale = pl.pallas_call(
    _scale_body,
    grid=(NP // BM,),
    in_specs=[_spec_rows, _spec_hist, _spec_hist, _spec_eye],
    out_specs=[_spec_rows, _spec_col, _spec_col, _spec_col],
    out_shape=[jax.ShapeDtypeStruct((NP, D), jnp.float32),
               _col_shape, _col_shape, _col_shape],
)

_mid = pl.pallas_call(
    _mid_body,
    grid=(NP // BM,),
    in_specs=[_spec_cat, _spec_rows, _spec_col, _spec_col, _spec_col,
              _spec_b, _spec_w],
    out_specs=_spec_rows,
    out_shape=jax.ShapeDtypeStruct((NP, D), jnp.float32),
)

_final = pl.pallas_call(
    _final_body,
    grid=(NP // BM,),
    in_specs=[_spec_cat, _spec_rows, _spec_col, _spec_col, _spec_col,
              _spec_b],
    out_specs=_spec_rows,
    out_shape=jax.ShapeDtypeStruct((N, D), jnp.float32),
)


@jax.jit
def kernel(x, edge_index, W1, b1, W2, b2):
    src = edge_index[0]
    dst = edge_index[1]
    npad = E_PAD - E
    padr = (jnp.arange(npad, dtype=jnp.int32) % (NP - N)) + N
    srcp = jnp.concatenate([src, padr])
    dstp = jnp.concatenate([dst, padr])
    src_prop = srcp.reshape(NTILE, NCH_S, 2 * CHUNK_P)
    dst_prop = dstp.reshape(NTILE, NCH_S, 2 * CHUNK_P)
    z1 = jnp.zeros((NP,), jnp.float32)
    z2 = jnp.zeros((NP, HALF), jnp.float32)

    cntp, loopp = _deg_call()(src_prop, dst_prop, z1)

    xw1 = _mm1(x, W1)
    y1, d3, dw3, di3 = _scale(xw1, cntp, loopp, jnp.eye(BM, dtype=jnp.float32))
    acc1 = _prop_call()(y1, src_prop, dst_prop, z2)
    y2 = _mid(acc1, y1, d3, dw3, di3, b1.reshape(1, D), W2)
    acc2 = _prop_call()(y2, src_prop, dst_prop, z2)
    return _final(acc2, y2, d3, dw3, di3, b2.reshape(1, D))
